# Initial kernel scaffold; baseline (speedup 1.0000x reference)
import jax
import jax.numpy as jnp
from jax.experimental import pallas as pl


def kernel(x, edge_index, x_prev, W, b, W_ih, W_hh, b_ih, b_hh):
    raise NotImplementedError("write your pallas kernel here")


# speedup vs baseline: 12.2849x; 12.2849x over previous
import functools

import jax
import jax.numpy as jnp
from jax import lax
from jax.experimental import pallas as pl
from jax.experimental.pallas import tpu as pltpu
from jax.experimental.pallas import tpu_sc as plsc

N = 10000
D = 128
E = 320000

NC = 2
NS = 16
NW = NC * NS
C = 128
CHUNKS = (E + NW * C - 1) // (NW * C)
E_PAD = CHUNKS * NW * C
N_ACC = 10112
ZROWS = 632
DEGW = 128

_mesh = functools.partial(
    plsc.VectorSubcoreMesh,
    core_axis_name="c", subcore_axis_name="s",
    num_cores=NC, num_subcores=NS,
)


def _sc_deg_body(dst_ref, degp_ref, didx, ones_v, zrow_v, acc, sem):
    cid = lax.axis_index("c")
    sid = lax.axis_index("s")
    wid = cid * NS + sid
    one16 = jnp.full((16,), 1.0, dtype=jnp.float32)
    zero16 = jnp.zeros((16,), dtype=jnp.float32)

    def fill(i, _):
        r = i // (DEGW // 16)
        c16 = (i % (DEGW // 16)) * 16
        ones_v[r, pl.ds(c16, 16)] = one16
        return 0
    lax.fori_loop(0, C * (DEGW // 16), fill, 0)

    def fill2(i, _):
        zrow_v[i // (DEGW // 16), pl.ds((i % (DEGW // 16)) * 16, 16)] = zero16
        return 0
    lax.fori_loop(0, 8 * (DEGW // 16), fill2, 0)

    def zcopy(q, _):
        pltpu.sync_copy(zrow_v, acc.at[pl.ds(sid * ZROWS + q * 8, 8)])
        return 0
    lax.fori_loop(0, ZROWS // 8, zcopy, 0)
    plsc.subcore_barrier()

    def chunk(j, _):
        base = (wid * CHUNKS + j) * C
        pltpu.sync_copy(dst_ref.at[pl.ds(base, C)], didx)
        pltpu.sync_copy(ones_v, acc.at[didx], add=True)
        return 0
    lax.fori_loop(0, CHUNKS, chunk, 0)
    plsc.subcore_barrier()

    pltpu.sync_copy(acc.at[pl.ds(sid * ZROWS, ZROWS)],
                    degp_ref.at[cid, pl.ds(sid * ZROWS, ZROWS)])


_sc_deg = pl.kernel(
    _sc_deg_body,
    out_type=jax.ShapeDtypeStruct((NC, N_ACC, DEGW), jnp.float32),
    mesh=_mesh(),
    scratch_types=[
        pltpu.VMEM((C,), jnp.int32),
        pltpu.VMEM((C, DEGW), jnp.float32),
        pltpu.VMEM((8, DEGW), jnp.float32),
        pltpu.VMEM_SHARED((N_ACC, DEGW), jnp.float32),
        pltpu.SemaphoreType.DMA,
    ],
)


def _sc_scatter_body(src_ref, dst_ref, hp_ref, sp_ref,
                     sidx, didx, rows_v, zrow_v, acc, sem):
    cid = lax.axis_index("c")
    sid = lax.axis_index("s")
    wid = cid * NS + sid
    zero16 = jnp.zeros((16,), dtype=jnp.float32)

    def fill(i, _):
        r = i // (D // 16)
        c16 = (i % (D // 16)) * 16
        zrow_v[r, pl.ds(c16, 16)] = zero16
        return 0
    lax.fori_loop(0, 8 * (D // 16), fill, 0)

    def zcopy(q, _):
        pltpu.sync_copy(zrow_v, acc.at[pl.ds(sid * ZROWS + q * 8, 8)])
        return 0
    lax.fori_loop(0, ZROWS // 8, zcopy, 0)
    plsc.subcore_barrier()

    def chunk(j, _):
        base = (wid * CHUNKS + j) * C
        pltpu.sync_copy(src_ref.at[pl.ds(base, C)], sidx)
        pltpu.sync_copy(dst_ref.at[pl.ds(base, C)], didx)
        pltpu.async_copy(hp_ref.at[sidx], rows_v, sem).wait()
        pltpu.sync_copy(rows_v, acc.at[didx], add=True)
        return 0
    lax.fori_loop(0, CHUNKS, chunk, 0)
    plsc.subcore_barrier()

    pltpu.sync_copy(acc.at[pl.ds(sid * ZROWS, ZROWS)],
                    sp_ref.at[cid, pl.ds(sid * ZROWS, ZROWS)])


_sc_scatter = pl.kernel(
    _sc_scatter_body,
    out_type=jax.ShapeDtypeStruct((NC, N_ACC, D), jnp.float32),
    mesh=_mesh(),
    scratch_types=[
        pltpu.VMEM((C,), jnp.int32),
        pltpu.VMEM((C,), jnp.int32),
        pltpu.VMEM((C, D), jnp.float32),
        pltpu.VMEM((8, D), jnp.float32),
        pltpu.VMEM_SHARED((N_ACC, D), jnp.float32),
        pltpu.SemaphoreType.DMA,
    ],
)

_BLK = 1000
_GRID = N // _BLK


def _tc_hp_body(x_ref, w_ref, degp_ref, hp_ref):
    deg = 1.0 + degp_ref[0, :, 0:1] + degp_ref[1, :, 0:1]
    dinv = lax.rsqrt(deg)
    h = jnp.dot(x_ref[...], w_ref[...], preferred_element_type=jnp.float32)
    hp_ref[...] = h * dinv


def _tc_hp(x, W, degp):
    return pl.pallas_call(
        _tc_hp_body,
        grid=(_GRID,),
        in_specs=[
            pl.BlockSpec((_BLK, D), lambda i: (i, 0)),
            pl.BlockSpec((D, D), lambda i: (0, 0)),
            pl.BlockSpec((NC, _BLK, DEGW), lambda i: (0, i, 0)),
        ],
        out_specs=pl.BlockSpec((_BLK, D), lambda i: (i, 0)),
        out_shape=jax.ShapeDtypeStruct((N, D), jnp.float32),
    )(x, W, degp)


def _tc_gru_body(sp_ref, degp_ref, hp_ref, xp_ref, wih_ref, whh_ref,
                 bih_ref, bhh_ref, b_ref, out_ref):
    deg = 1.0 + degp_ref[0, :, 0:1] + degp_ref[1, :, 0:1]
    dinv = lax.rsqrt(deg)
    S = sp_ref[0] + sp_ref[1]
    hp = hp_ref[...]
    gcn = dinv * (S + hp) + b_ref[...]
    xp = xp_ref[...]
    gi = jnp.dot(gcn, wih_ref[...], preferred_element_type=jnp.float32) + bih_ref[...]
    gh = jnp.dot(xp, whh_ref[...], preferred_element_type=jnp.float32) + bhh_ref[...]
    r = jax.nn.sigmoid(gi[:, :D] + gh[:, :D])
    z = jax.nn.sigmoid(gi[:, D:2 * D] + gh[:, D:2 * D])
    n = jnp.tanh(gi[:, 2 * D:] + r * gh[:, 2 * D:])
    out_ref[...] = (1.0 - z) * n + z * xp


def _tc_gru(sp, degp, hp, x_prev, W_ih_T, W_hh_T, b_ih, b_hh, b):
    return pl.pallas_call(
        _tc_gru_body,
        grid=(_GRID,),
        in_specs=[
            pl.BlockSpec((NC, _BLK, D), lambda i: (0, i, 0)),
            pl.BlockSpec((NC, _BLK, DEGW), lambda i: (0, i, 0)),
            pl.BlockSpec((_BLK, D), lambda i: (i, 0)),
            pl.BlockSpec((_BLK, D), lambda i: (i, 0)),
            pl.BlockSpec((D, 3 * D), lambda i: (0, 0)),
            pl.BlockSpec((D, 3 * D), lambda i: (0, 0)),
            pl.BlockSpec((1, 3 * D), lambda i: (0, 0)),
            pl.BlockSpec((1, 3 * D), lambda i: (0, 0)),
            pl.BlockSpec((1, D), lambda i: (0, 0)),
        ],
        out_specs=pl.BlockSpec((_BLK, D), lambda i: (i, 0)),
        out_shape=jax.ShapeDtypeStruct((N, D), jnp.float32),
    )(sp, degp, hp, x_prev, W_ih_T, W_hh_T, b_ih, b_hh, b)


def kernel(x, edge_index, x_prev, W, b, W_ih, W_hh, b_ih, b_hh):
    src = edge_index[0].astype(jnp.int32)
    dst = edge_index[1].astype(jnp.int32)
    src = jnp.concatenate([src, jnp.zeros((E_PAD - E,), jnp.int32)])
    dst = jnp.concatenate([dst, jnp.full((E_PAD - E,), N, jnp.int32)])

    degp = _sc_deg(dst)
    hp = _tc_hp(x, W, degp)
    sp = _sc_scatter(src, dst, hp)
    return _tc_gru(sp, degp, hp, x_prev,
                   W_ih.T, W_hh.T,
                   b_ih.reshape(1, 3 * D), b_hh.reshape(1, 3 * D),
                   b.reshape(1, D))
